# baseline (speedup 1.0000x reference)
import jax
import jax.numpy as jnp
from jax.experimental import pallas as pl
from jax.experimental.pallas import tpu as pltpu

N_CODEBOOKS = 4
K = 1024
D = 256
BATCH = 16384
BETA = 0.25
BLK = 512
GRID = BATCH // BLK


def _rvq_body(x_ref, e0_ref, e1_ref, e2_ref, e3_ref,
              q_ref, codes_ref, loss_ref, unused_ref, counts_ref):
    step = pl.program_id(0)

    @pl.when(step == 0)
    def _init():
        counts_ref[...] = jnp.zeros_like(counts_ref)
        loss_ref[...] = jnp.zeros_like(loss_ref)
        unused_ref[...] = jnp.zeros_like(unused_ref)

    x = x_ref[...]
    r = x
    q = jnp.zeros_like(x)
    loss = jnp.float32(0.0)
    codes = []
    for s, t_ref in enumerate((e0_ref, e1_ref, e2_ref, e3_ref)):
        t = t_ref[...]
        rn = jnp.sum(r * r, axis=1, keepdims=True)
        en = jnp.sum(t * t, axis=1, keepdims=True).T
        m = jax.lax.dot_general(r, t, (((1,), (1,)), ((), ())),
                                preferred_element_type=jnp.float32)
        dist = (rn - 2.0 * m) + en
        minv = jnp.min(dist, axis=1, keepdims=True)
        iota = jax.lax.broadcasted_iota(jnp.int32, dist.shape, 1)
        ind = jnp.min(jnp.where(dist == minv, iota, K),
                      axis=1, keepdims=True)
        oh = (iota == ind).astype(jnp.float32)
        xq = jax.lax.dot_general(oh, t, (((1,), (0,)), ((), ())),
                                 preferred_element_type=jnp.float32,
                                 precision=jax.lax.Precision.HIGHEST)
        counts_ref[s:s + 1, :] += jnp.sum(oh, axis=0, keepdims=True)
        loss = loss + jnp.sum(minv)
        q = q + xq
        r = r - xq
        codes.append(ind)

    q_ref[...] = q
    codes_ref[...] = jnp.concatenate(codes, axis=1)
    loss_ref[...] = loss_ref[...] + loss

    @pl.when(step == GRID - 1)
    def _finalize():
        loss_ref[...] = (loss_ref[...] * (1.0 + BETA)
                         / (N_CODEBOOKS * BATCH * D))
        unused_ref[...] = jnp.sum(
            (counts_ref[...] == 0.0).astype(jnp.int32),
            axis=(0, 1), keepdims=True)


def kernel(x, embed_0, embed_1, embed_2, embed_3):
    table_spec = pl.BlockSpec((K, D), lambda i: (0, 0))
    q, codes, loss, unused, _counts = pl.pallas_call(
        _rvq_body,
        grid=(GRID,),
        in_specs=[
            pl.BlockSpec((BLK, D), lambda i: (i, 0)),
            table_spec, table_spec, table_spec, table_spec,
        ],
        out_specs=[
            pl.BlockSpec((BLK, D), lambda i: (i, 0)),
            pl.BlockSpec((BLK, N_CODEBOOKS), lambda i: (i, 0)),
            pl.BlockSpec((1, 1), lambda i: (0, 0)),
            pl.BlockSpec((1, 1), lambda i: (0, 0)),
            pl.BlockSpec((N_CODEBOOKS, K), lambda i: (0, 0)),
        ],
        out_shape=[
            jax.ShapeDtypeStruct((BATCH, D), jnp.float32),
            jax.ShapeDtypeStruct((BATCH, N_CODEBOOKS), jnp.int32),
            jax.ShapeDtypeStruct((1, 1), jnp.float32),
            jax.ShapeDtypeStruct((1, 1), jnp.int32),
            jax.ShapeDtypeStruct((N_CODEBOOKS, K), jnp.float32),
        ],
        compiler_params=pltpu.CompilerParams(
            dimension_semantics=("arbitrary",)),
    )(x, embed_0, embed_1, embed_2, embed_3)
    return q, loss.reshape(()), unused.reshape(()), codes


# speedup vs baseline: 1.5358x; 1.5358x over previous
import jax
import jax.numpy as jnp
from jax.experimental import pallas as pl
from jax.experimental.pallas import tpu as pltpu

N_CODEBOOKS = 4
K = 1024
D = 256
BATCH = 16384
BETA = 0.25
BLK = 512
GRID = BATCH // BLK


def _rvq_body(x_ref, e0_ref, e1_ref, e2_ref, e3_ref,
              q_ref, codes_ref, loss_ref, unused_ref, counts_ref,
              *split_refs):
    step = pl.program_id(0)
    table_refs = (e0_ref, e1_ref, e2_ref, e3_ref)

    @pl.when(step == 0)
    def _init():
        counts_ref[...] = jnp.zeros_like(counts_ref)
        loss_ref[...] = jnp.zeros_like(loss_ref)
        unused_ref[...] = jnp.zeros_like(unused_ref)
        for s in range(N_CODEBOOKS):
            t = table_refs[s][...]
            t1 = t.astype(jnp.bfloat16)
            r1 = t - t1.astype(jnp.float32)
            t2 = r1.astype(jnp.bfloat16)
            r2 = r1 - t2.astype(jnp.float32)
            t3 = r2.astype(jnp.bfloat16)
            split_refs[3 * s][...] = t1
            split_refs[3 * s + 1][...] = t2
            split_refs[3 * s + 2][...] = t3

    x = x_ref[...]
    r = x
    q = jnp.zeros_like(x)
    loss = jnp.float32(0.0)
    codes = []
    for s, t_ref in enumerate(table_refs):
        t = t_ref[...]
        rn = jnp.sum(r * r, axis=1, keepdims=True)
        en = jnp.sum(t * t, axis=1, keepdims=True).T
        m = jax.lax.dot_general(r, t, (((1,), (1,)), ((), ())),
                                preferred_element_type=jnp.float32)
        dist = (rn - 2.0 * m) + en
        minv = jnp.min(dist, axis=1, keepdims=True)
        iota = jax.lax.broadcasted_iota(jnp.int32, dist.shape, 1)
        ind = jnp.min(jnp.where(dist == minv, iota, K),
                      axis=1, keepdims=True)
        hit = iota == ind
        oh16 = hit.astype(jnp.bfloat16)
        dims = (((1,), (0,)), ((), ()))
        xq = ((jax.lax.dot_general(oh16, split_refs[3 * s][...], dims,
                                   preferred_element_type=jnp.float32)
               + jax.lax.dot_general(oh16, split_refs[3 * s + 1][...], dims,
                                     preferred_element_type=jnp.float32))
              + jax.lax.dot_general(oh16, split_refs[3 * s + 2][...], dims,
                                    preferred_element_type=jnp.float32))
        counts_ref[s:s + 1, :] += jnp.sum(hit.astype(jnp.float32),
                                          axis=0, keepdims=True)
        loss = loss + jnp.sum(minv)
        q = q + xq
        r = r - xq
        codes.append(ind)

    q_ref[...] = q
    codes_ref[...] = jnp.concatenate(codes, axis=1)
    loss_ref[...] = loss_ref[...] + loss

    @pl.when(step == GRID - 1)
    def _finalize():
        loss_ref[...] = (loss_ref[...] * (1.0 + BETA)
                         / (N_CODEBOOKS * BATCH * D))
        unused_ref[...] = jnp.sum(
            (counts_ref[...] == 0.0).astype(jnp.int32),
            axis=(0, 1), keepdims=True)


def kernel(x, embed_0, embed_1, embed_2, embed_3):
    table_spec = pl.BlockSpec((K, D), lambda i: (0, 0))
    q, codes, loss, unused, _counts = pl.pallas_call(
        _rvq_body,
        grid=(GRID,),
        in_specs=[
            pl.BlockSpec((BLK, D), lambda i: (i, 0)),
            table_spec, table_spec, table_spec, table_spec,
        ],
        out_specs=[
            pl.BlockSpec((BLK, D), lambda i: (i, 0)),
            pl.BlockSpec((BLK, N_CODEBOOKS), lambda i: (i, 0)),
            pl.BlockSpec((1, 1), lambda i: (0, 0)),
            pl.BlockSpec((1, 1), lambda i: (0, 0)),
            pl.BlockSpec((N_CODEBOOKS, K), lambda i: (0, 0)),
        ],
        out_shape=[
            jax.ShapeDtypeStruct((BATCH, D), jnp.float32),
            jax.ShapeDtypeStruct((BATCH, N_CODEBOOKS), jnp.int32),
            jax.ShapeDtypeStruct((1, 1), jnp.float32),
            jax.ShapeDtypeStruct((1, 1), jnp.int32),
            jax.ShapeDtypeStruct((N_CODEBOOKS, K), jnp.float32),
        ],
        scratch_shapes=[pltpu.VMEM((K, D), jnp.bfloat16)] * 12,
        compiler_params=pltpu.CompilerParams(
            dimension_semantics=("arbitrary",)),
    )(x, embed_0, embed_1, embed_2, embed_3)
    return q, loss.reshape(()), unused.reshape(()), codes
